# baseline (speedup 1.0000x reference)
import functools

import jax
import jax.numpy as jnp
from jax import lax
from jax.experimental import pallas as pl
from jax.experimental.pallas import tpu as pltpu
from jax.experimental.pallas import tpu_sc as plsc

L = 16
PSH = 14


@jax.jit
def kernel(latent_id, latents):
    B, = latent_id.shape
    V, D = latents.shape
    NC, NS = 2, 16
    HALF = B // NC
    PPT = HALF // NS
    CH = 256
    VMAIN = (V // 128) * 128
    NFULL = VMAIN // CH
    SHORT = VMAIN - NFULL * CH
    TAIL = V - VMAIN
    PB = PPT // 4

    tableT = latents.T
    tail_lin = latents[VMAIN:, :].reshape(-1)
    idx32 = latent_id.astype(jnp.int32)

    @functools.partial(
        pl.kernel,
        out_type=jax.ShapeDtypeStruct((D, B), jnp.float32),
        mesh=plsc.VectorSubcoreMesh(core_axis_name="c", subcore_axis_name="s"),
        scratch_types=[
            pltpu.VMEM((HALF,), jnp.int32),
            pltpu.VMEM((HALF + L,), jnp.int32),
            pltpu.VMEM((HALF + L,), jnp.int32),
            pltpu.VMEM((D, CH), jnp.float32),
            pltpu.VMEM((TAIL * D,), jnp.float32),
            pltpu.VMEM((L, 2 * D), jnp.float32),
            pltpu.VMEM((L,), jnp.int32),
            pltpu.VMEM((PB, 2 * D), jnp.float32),
            pltpu.VMEM((PB,), jnp.int32),
            pltpu.VMEM_SHARED((HALF + L, 2 * D), jnp.float32),
            pltpu.SemaphoreType.DMA,
        ],
        compiler_params=pltpu.CompilerParams(use_tc_tiling_on_sc=True,
                                             needs_layout_passes=False),
    )
    def run(idx_hbm, tT_hbm, tail_hbm, outT_hbm,
            idx_v, m_key, cm_key, buf, tailb, block, posb, rowbuf, rowidx,
            stage, sem):
        cid = lax.axis_index("c")
        sid = lax.axis_index("s")
        iota = lax.iota(jnp.int32, L)

        def splat(x):
            return jnp.full((L,), 0, jnp.int32) + x

        start_ch = sid * 24 + jnp.minimum(sid, 6)
        end_ch = (sid + 1) * 24 + jnp.minimum(sid + 1, 6)
        lo = start_ch * CH
        hi = jnp.where(sid == NS - 1, V, end_ch * CH)

        pltpu.sync_copy(idx_hbm.at[pl.ds(pl.multiple_of(cid * HALF, HALF),
                                         HALF)], idx_v)
        pltpu.sync_copy(tail_hbm, tailb)

        lov = splat(lo)
        hiv = splat(hi)

        def scan_body(k, cnt):
            v = idx_v[pl.ds(k * L, L)]
            key = (v << PSH) | (iota + k * L)
            m = (v >= lov) & (v < hiv)
            plsc.store_compressed(m_key.at[pl.ds(cnt, L)], key, mask=m)
            return cnt + jnp.sum(m.astype(jnp.int32))

        cnt = lax.fori_loop(0, HALF // L, scan_body, jnp.int32(0))
        m_key[pl.ds(cnt, L)] = jnp.full((L,), jnp.int32(0x7FFFFFFF))
        nmg = (cnt + (L - 1)) // L

        def process(clo, cext, gather_fn):
            clov = splat(clo << PSH)
            chiv = splat((clo + cext) << PSH)

            def cbody(g, cc):
                key = m_key[pl.ds(g * L, L)]
                m = (key >= clov) & (key < chiv)
                plsc.store_compressed(cm_key.at[pl.ds(cc, L)],
                                      key - clov, mask=m)
                return cc + jnp.sum(m.astype(jnp.int32))

            cc = lax.fori_loop(0, nmg, cbody, jnp.int32(0))
            cm_key[pl.ds(cc, L)] = jnp.full((L,), HALF, jnp.int32)

            def ebody(g, carry):
                ckey = cm_key[pl.ds(g * L, L)]
                jrel = ckey >> PSH
                pos = ckey & ((1 << PSH) - 1)
                posb[...] = pos
                for c in range(D):
                    vals = gather_fn(c, jrel)
                    plsc.store_scatter(block, [iota, splat(c)], vals)
                pltpu.sync_copy(block, stage.at[posb])
                return carry

            lax.fori_loop(0, (cc + (L - 1)) // L, ebody, jnp.int32(0))

        def chunk_gather(c, jrel):
            return plsc.load_gather(buf, [splat(c), jrel])

        def tail_gather(c, jrel):
            return plsc.load_gather(tailb, [jrel * D + c])

        def chunk_body(ch, carry):
            clo = pl.multiple_of(ch * CH, CH)
            pltpu.sync_copy(tT_hbm.at[:, pl.ds(clo, CH)], buf)
            process(clo, CH, chunk_gather)
            return carry

        lax.fori_loop(start_ch, jnp.minimum(end_ch, NFULL), chunk_body,
                      jnp.int32(0))

        @pl.when(sid == NS - 1)
        def _():
            pltpu.sync_copy(tT_hbm.at[:, pl.ds(NFULL * CH, SHORT)],
                            buf.at[:, pl.ds(0, SHORT)])
            process(NFULL * CH, SHORT, chunk_gather)
            process(VMAIN, TAIL, tail_gather)

        plsc.subcore_barrier()
        for sub in range(PPT // PB):
            pbase = sid * PPT + sub * PB

            def ibody(r, carry):
                rowidx[pl.ds(r * L, L)] = iota + (pbase + r * L)
                return carry

            lax.fori_loop(0, PB // L, ibody, jnp.int32(0))
            pltpu.async_copy(stage.at[rowidx], rowbuf, sem).wait()

            def tbody(g, carry):
                rows = iota + g * L
                for c in range(D):
                    vals = plsc.load_gather(rowbuf, [rows, splat(c)])
                    buf[c, pl.ds(g * L, L)] = vals
                return carry

            lax.fori_loop(0, PB // L, tbody, jnp.int32(0))
            pltpu.sync_copy(buf.at[:, pl.ds(0, PB)],
                            outT_hbm.at[:, pl.ds(cid * HALF + pbase, PB)])

    return run(idx32, tableT, tail_lin).T


# speedup vs baseline: 1.4723x; 1.4723x over previous
import functools

import jax
import jax.numpy as jnp
from jax import lax
from jax.experimental import pallas as pl
from jax.experimental.pallas import tpu as pltpu
from jax.experimental.pallas import tpu_sc as plsc


@jax.jit
def kernel(latent_id, latents):
    B, = latent_id.shape
    V, D = latents.shape
    info = plsc.get_sparse_core_info()
    NC, NS = info.num_cores, info.num_subcores
    NW = NC * NS
    assert B % NW == 0
    b_per_w = B // NW

    @functools.partial(
        pl.kernel,
        out_type=jax.ShapeDtypeStruct((B, D), latents.dtype),
        mesh=plsc.VectorSubcoreMesh(core_axis_name="c", subcore_axis_name="s"),
        scratch_types=[
            pltpu.VMEM((b_per_w,), jnp.int32),
            pltpu.VMEM((b_per_w, D), latents.dtype),
            pltpu.SemaphoreType.DMA,
        ],
        compiler_params=pltpu.CompilerParams(use_tc_tiling_on_sc=False),
    )
    def run(idx_hbm, table_hbm, out_hbm, idx_v, rows_v, sem):
        wid = lax.axis_index("s") * NC + lax.axis_index("c")
        base = wid * b_per_w
        pltpu.sync_copy(idx_hbm.at[pl.ds(base, b_per_w)], idx_v)
        pltpu.async_copy(table_hbm.at[idx_v], rows_v, sem).wait()
        pltpu.sync_copy(rows_v, out_hbm.at[pl.ds(base, b_per_w)])

    return run(latent_id.astype(jnp.int32), latents)
